# baseline (speedup 1.0000x reference)
import functools

import jax
import jax.numpy as jnp
from jax import lax
from jax.experimental import pallas as pl
from jax.experimental.pallas import tpu as pltpu
from jax.experimental.pallas import tpu_sc as plsc

N = 10000
E = 320000
D = 128
DH = D // 2

NCORES = 2
NSUB = 16
EPT = E // NSUB
C = 125
NCH = EPT // C
NBUF = 4
IDEPTH = 8
SPF = 5
RPT = 624
TAIL = N - NSUB * RPT


def _mm_body(x_ref, w_ref, h_ref):
    h_ref[0] = lax.dot_general(
        x_ref[...], w_ref[...],
        (((1,), (1,)), ((), ())),
        preferred_element_type=jnp.float32,
    )


def _matmul(x, W):
    return pl.pallas_call(
        _mm_body,
        grid=(2, 10),
        in_specs=[
            pl.BlockSpec((N // 10, D), lambda k, i: (i, 0)),
            pl.BlockSpec((DH, D), lambda k, i: (k, 0)),
        ],
        out_specs=pl.BlockSpec((1, N // 10, DH), lambda k, i: (k, i, 0)),
        out_shape=jax.ShapeDtypeStruct((2, N, DH), jnp.float32),
    )(x, W)


def _cat_body(p_ref, o_ref):
    o_ref[...] = jnp.concatenate([p_ref[0], p_ref[1]], axis=-1)


def _assemble(p):
    return pl.pallas_call(
        _cat_body,
        grid=(10,),
        in_specs=[pl.BlockSpec((2, N // 10, DH), lambda i: (0, i, 0))],
        out_specs=pl.BlockSpec((N // 10, D), lambda i: (i, 0)),
        out_shape=jax.ShapeDtypeStruct((N, D), jnp.float32),
    )(p)


_mesh = plsc.VectorSubcoreMesh(core_axis_name="c", subcore_axis_name="s")


@functools.partial(
    pl.kernel,
    mesh=_mesh,
    compiler_params=pltpu.CompilerParams(use_tc_tiling_on_sc=False),
    out_type=jax.ShapeDtypeStruct((N, D), jnp.float32),
    scratch_types=(
        [
            pltpu.VMEM((IDEPTH, C), jnp.int32),
            pltpu.VMEM((IDEPTH, C), jnp.int32),
            pltpu.VMEM((C, DH), jnp.float32),
            pltpu.VMEM((C, DH), jnp.float32),
            pltpu.VMEM((C, DH), jnp.float32),
            pltpu.VMEM((C, DH), jnp.float32),
            pltpu.VMEM_SHARED((N, DH), jnp.float32),
            pltpu.VMEM_SHARED((N, DH), jnp.float32),
        ]
        + [pltpu.SemaphoreType.DMA] * (NBUF + NBUF + IDEPTH + 1)
    ),
)
def _scatter_kernel(h_hbm, row_hbm, col_hbm, out_hbm,
                    rbufs, cbufs, g0, g1, g2, g3, hcopy, acc, *sems):
    c = lax.axis_index("c")
    s = lax.axis_index("s")
    base_r = s * RPT
    g = [g0, g1, g2, g3]
    gsem = sems[0:NBUF]
    ssem = sems[NBUF:2 * NBUF]
    isem = sems[2 * NBUF:2 * NBUF + IDEPTH]
    hsem = sems[2 * NBUF + IDEPTH]

    hcp = [pltpu.async_copy(h_hbm.at[c].at[pl.ds(base_r, RPT)],
                            hcopy.at[pl.ds(base_r, RPT)], hsem)]

    @pl.when(s == NSUB - 1)
    def _htail():
        pltpu.async_copy(h_hbm.at[c].at[pl.ds(NSUB * RPT, TAIL)],
                         hcopy.at[pl.ds(NSUB * RPT, TAIL)], hsem).wait()

    def _idx_issue(j, slot):
        pltpu.async_copy(row_hbm.at[s, j], rbufs.at[slot], isem[slot])
        pltpu.async_copy(col_hbm.at[s, j], cbufs.at[slot], isem[slot])

    def _idx_wait(j, slot):
        pltpu.make_async_copy(row_hbm.at[s, j], rbufs.at[slot],
                              isem[slot]).wait()
        pltpu.make_async_copy(col_hbm.at[s, j], cbufs.at[slot],
                              isem[slot]).wait()

    for slot in range(IDEPTH):
        _idx_issue(slot, slot)

    def _zrow(i, carry):
        for b in range(NBUF):
            for t in range(DH // 16):
                g[b][i, pl.ds(t * 16, 16)] = jnp.zeros((16,), jnp.float32)
        return carry
    lax.fori_loop(0, C, _zrow, 0)

    zcp = []
    for k in range(5):
        nr = 124 if k == 4 else 125
        zcp.append(pltpu.async_copy(
            g[k % NBUF].at[pl.ds(0, nr)],
            acc.at[pl.ds(base_r + k * 125, nr)],
            ssem[k % NBUF]))

    @pl.when(s == NSUB - 1)
    def _ztail():
        pltpu.async_copy(g[0].at[pl.ds(0, TAIL)],
                         acc.at[pl.ds(NSUB * RPT, TAIL)], ssem[0]).wait()

    for cp in zcp:
        cp.wait()
    for cp in hcp:
        cp.wait()

    plsc.subcore_barrier()

    def _gather_spmem(slot, b):
        pltpu.async_copy(hcopy.at[cbufs.at[slot]], g[b], gsem[b])

    def _gather_hbm(slot, b):
        pltpu.async_copy(h_hbm.at[c].at[cbufs.at[slot]], g[b], gsem[b])

    def _gather_wait(slot, b):
        pltpu.make_async_copy(hcopy.at[cbufs.at[slot]], g[b],
                              gsem[b]).wait()

    for b in range(NBUF):
        _idx_wait(b, b)
        _gather_spmem(b, b)

    def _sgroup(sg, carry):
        p = lax.rem(sg, 2)
        for half in range(2):
            j0 = sg * 8 + half * 4
            for b in range(NBUF):
                slot_j = half * 4 + b
                _gather_wait(slot_j, b)
                pltpu.async_copy(g[b], acc.at[rbufs.at[slot_j]],
                                 ssem[b], add=True)
            for b in range(NBUF):
                j = j0 + b
                slot_j = half * 4 + b
                slot_n = (slot_j + 4) % 8
                pltpu.make_async_copy(g[b], acc.at[rbufs.at[slot_j]],
                                      ssem[b]).wait()
                nxt = j + NBUF
                inb = nxt < NCH

                @pl.when(inb)
                def _widx():
                    _idx_wait(nxt, slot_n)

                if half == 0:
                    if b == 0:
                        spm = (p == 0)

                        @pl.when(inb & spm)
                        def _gs():
                            _gather_spmem(slot_n, b)

                        @pl.when(inb & jnp.logical_not(spm))
                        def _gh():
                            _gather_hbm(slot_n, b)
                    else:
                        @pl.when(inb)
                        def _gh2():
                            _gather_hbm(slot_n, b)
                else:
                    spm = (p == 1)

                    @pl.when(inb & spm)
                    def _gs3():
                        _gather_spmem(slot_n, b)

                    @pl.when(inb & jnp.logical_not(spm))
                    def _gh4():
                        _gather_hbm(slot_n, b)

                nxt8 = j + IDEPTH

                @pl.when(nxt8 < NCH)
                def _load_idx():
                    _idx_issue(nxt8, slot_j)
        return carry
    lax.fori_loop(0, NCH // 8, _sgroup, 0)

    plsc.subcore_barrier()

    pltpu.sync_copy(acc.at[pl.ds(base_r, RPT)],
                    out_hbm.at[pl.ds(base_r, RPT), pl.ds(c * DH, DH)])

    @pl.when(s == NSUB - 1)
    def _wtail():
        pltpu.sync_copy(acc.at[pl.ds(NSUB * RPT, TAIL)],
                        out_hbm.at[pl.ds(NSUB * RPT, TAIL), pl.ds(c * DH, DH)])


def kernel(x, edge_index, W):
    h2 = _matmul(x, W)
    row3d = edge_index[0].reshape(NSUB, NCH, C)
    col3d = edge_index[1].reshape(NSUB, NCH, C)
    return _scatter_kernel(h2, row3d, col3d)


# speedup vs baseline: 1.1759x; 1.1759x over previous
import functools

import jax
import jax.numpy as jnp
from jax import lax
from jax.experimental import pallas as pl
from jax.experimental.pallas import tpu as pltpu
from jax.experimental.pallas import tpu_sc as plsc

N = 10000
E = 320000
D = 128
DH = D // 2

NCORES = 2
NSUB = 16
EPT = E // NSUB
C = 125
NCH = EPT // C
NBUF = 4
IDEPTH = 8
SPF = 5
RPT = 624
TAIL = N - NSUB * RPT


def _mm_body(x_ref, w_ref, h_ref):
    h_ref[0] = lax.dot_general(
        x_ref[...], w_ref[...],
        (((1,), (1,)), ((), ())),
        preferred_element_type=jnp.float32,
    )


def _matmul(x, W):
    return pl.pallas_call(
        _mm_body,
        grid=(2, 10),
        in_specs=[
            pl.BlockSpec((N // 10, D), lambda k, i: (i, 0)),
            pl.BlockSpec((DH, D), lambda k, i: (k, 0)),
        ],
        out_specs=pl.BlockSpec((1, N // 10, DH), lambda k, i: (k, i, 0)),
        out_shape=jax.ShapeDtypeStruct((2, N, DH), jnp.float32),
    )(x, W)


def _cat_body(p_ref, o_ref):
    o_ref[...] = jnp.concatenate([p_ref[0], p_ref[1]], axis=-1)


def _assemble(p):
    return pl.pallas_call(
        _cat_body,
        grid=(10,),
        in_specs=[pl.BlockSpec((2, N // 10, DH), lambda i: (0, i, 0))],
        out_specs=pl.BlockSpec((N // 10, D), lambda i: (i, 0)),
        out_shape=jax.ShapeDtypeStruct((N, D), jnp.float32),
    )(p)


_mesh = plsc.VectorSubcoreMesh(core_axis_name="c", subcore_axis_name="s")


@functools.partial(
    pl.kernel,
    mesh=_mesh,
    compiler_params=pltpu.CompilerParams(use_tc_tiling_on_sc=False),
    out_type=jax.ShapeDtypeStruct((N, D), jnp.float32),
    scratch_types=(
        [
            pltpu.VMEM((IDEPTH, C), jnp.int32),
            pltpu.VMEM((IDEPTH, C), jnp.int32),
            pltpu.VMEM((C, DH), jnp.float32),
            pltpu.VMEM((C, DH), jnp.float32),
            pltpu.VMEM((C, DH), jnp.float32),
            pltpu.VMEM((C, DH), jnp.float32),
            pltpu.VMEM_SHARED((N, DH), jnp.float32),
            pltpu.VMEM_SHARED((N, DH), jnp.float32),
        ]
        + [pltpu.SemaphoreType.DMA] * (NBUF + NBUF + IDEPTH + 1)
    ),
)
def _scatter_kernel(h_hbm, row_hbm, col_hbm, out_hbm,
                    rbufs, cbufs, g0, g1, g2, g3, hcopy, acc, *sems):
    c = lax.axis_index("c")
    s = lax.axis_index("s")
    base_r = s * RPT
    g = [g0, g1, g2, g3]
    gsem = sems[0:NBUF]
    ssem = sems[NBUF:2 * NBUF]
    isem = sems[2 * NBUF:2 * NBUF + IDEPTH]
    hsem = sems[2 * NBUF + IDEPTH]

    hcp = [pltpu.async_copy(h_hbm.at[c].at[pl.ds(base_r, RPT)],
                            hcopy.at[pl.ds(base_r, RPT)], hsem)]

    @pl.when(s == NSUB - 1)
    def _htail():
        pltpu.async_copy(h_hbm.at[c].at[pl.ds(NSUB * RPT, TAIL)],
                         hcopy.at[pl.ds(NSUB * RPT, TAIL)], hsem).wait()

    def _idx_issue(j, slot):
        pltpu.async_copy(row_hbm.at[s, j], rbufs.at[slot], isem[slot])
        pltpu.async_copy(col_hbm.at[s, j], cbufs.at[slot], isem[slot])

    def _idx_wait(j, slot):
        pltpu.make_async_copy(row_hbm.at[s, j], rbufs.at[slot],
                              isem[slot]).wait()
        pltpu.make_async_copy(col_hbm.at[s, j], cbufs.at[slot],
                              isem[slot]).wait()

    for slot in range(IDEPTH):
        _idx_issue(slot, slot)

    def _zrow(i, carry):
        for b in range(NBUF):
            for t in range(DH // 16):
                g[b][i, pl.ds(t * 16, 16)] = jnp.zeros((16,), jnp.float32)
        return carry
    lax.fori_loop(0, C, _zrow, 0)

    zcp = []
    for k in range(5):
        nr = 124 if k == 4 else 125
        zcp.append(pltpu.async_copy(
            g[k % NBUF].at[pl.ds(0, nr)],
            acc.at[pl.ds(base_r + k * 125, nr)],
            ssem[k % NBUF]))

    @pl.when(s == NSUB - 1)
    def _ztail():
        pltpu.async_copy(g[0].at[pl.ds(0, TAIL)],
                         acc.at[pl.ds(NSUB * RPT, TAIL)], ssem[0]).wait()

    for cp in zcp:
        cp.wait()
    for cp in hcp:
        cp.wait()

    plsc.subcore_barrier()

    def _gather_spmem(slot, b):
        pltpu.async_copy(hcopy.at[cbufs.at[slot]], g[b], gsem[b])

    def _gather_hbm(slot, b):
        pltpu.async_copy(h_hbm.at[c].at[cbufs.at[slot]], g[b], gsem[b])

    def _gather_wait(slot, b):
        pltpu.make_async_copy(hcopy.at[cbufs.at[slot]], g[b],
                              gsem[b]).wait()

    for b in range(NBUF):
        _idx_wait(b, b)
        _gather_spmem(b, b)

    def _sgroup(sg, carry):
        p = lax.rem(sg, 2)
        for half in range(2):
            j0 = sg * 8 + half * 4
            for b in range(NBUF):
                slot_j = half * 4 + b
                _gather_wait(slot_j, b)
                pltpu.async_copy(g[b], acc.at[rbufs.at[slot_j]],
                                 ssem[b], add=True)
            for b in range(NBUF):
                j = j0 + b
                slot_j = half * 4 + b
                slot_n = (slot_j + 4) % 8
                pltpu.make_async_copy(g[b], acc.at[rbufs.at[slot_j]],
                                      ssem[b]).wait()
                nxt = j + NBUF
                inb = nxt < NCH

                @pl.when(inb)
                def _widx():
                    _idx_wait(nxt, slot_n)

                @pl.when(inb)
                def _gh_only():
                    _gather_hbm(slot_n, b)

                nxt8 = j + IDEPTH

                @pl.when(nxt8 < NCH)
                def _load_idx():
                    _idx_issue(nxt8, slot_j)
        return carry
    lax.fori_loop(0, NCH // 8, _sgroup, 0)

    plsc.subcore_barrier()

    pltpu.sync_copy(acc.at[pl.ds(base_r, RPT)],
                    out_hbm.at[pl.ds(base_r, RPT), pl.ds(c * DH, DH)])

    @pl.when(s == NSUB - 1)
    def _wtail():
        pltpu.sync_copy(acc.at[pl.ds(NSUB * RPT, TAIL)],
                        out_hbm.at[pl.ds(NSUB * RPT, TAIL), pl.ds(c * DH, DH)])


def kernel(x, edge_index, W):
    h2 = _matmul(x, W)
    row3d = edge_index[0].reshape(NSUB, NCH, C)
    col3d = edge_index[1].reshape(NSUB, NCH, C)
    return _scatter_kernel(h2, row3d, col3d)


# speedup vs baseline: 1.2115x; 1.0303x over previous
import functools

import jax
import jax.numpy as jnp
from jax import lax
from jax.experimental import pallas as pl
from jax.experimental.pallas import tpu as pltpu
from jax.experimental.pallas import tpu_sc as plsc

N = 10000
E = 320000
D = 128
DH = D // 2

NCORES = 2
NSUB = 16
EPT = E // NSUB
C = 125
NCH = EPT // C
NBUF = 4
RPT = 624
TAIL = N - NSUB * RPT


def _mm_body(x_ref, w_ref, h_ref):
    h_ref[0] = lax.dot_general(
        x_ref[...], w_ref[...],
        (((1,), (1,)), ((), ())),
        preferred_element_type=jnp.float32,
    )


def _matmul(x, W):
    return pl.pallas_call(
        _mm_body,
        grid=(2, 10),
        in_specs=[
            pl.BlockSpec((N // 10, D), lambda k, i: (i, 0)),
            pl.BlockSpec((DH, D), lambda k, i: (k, 0)),
        ],
        out_specs=pl.BlockSpec((1, N // 10, DH), lambda k, i: (k, i, 0)),
        out_shape=jax.ShapeDtypeStruct((2, N, DH), jnp.float32),
    )(x, W)


def _cat_body(p_ref, o_ref):
    o_ref[...] = jnp.concatenate([p_ref[0], p_ref[1]], axis=-1)


def _assemble(p):
    return pl.pallas_call(
        _cat_body,
        grid=(10,),
        in_specs=[pl.BlockSpec((2, N // 10, DH), lambda i: (0, i, 0))],
        out_specs=pl.BlockSpec((N // 10, D), lambda i: (i, 0)),
        out_shape=jax.ShapeDtypeStruct((N, D), jnp.float32),
    )(p)


_mesh = plsc.VectorSubcoreMesh(core_axis_name="c", subcore_axis_name="s")


@functools.partial(
    pl.kernel,
    mesh=_mesh,
    compiler_params=pltpu.CompilerParams(use_tc_tiling_on_sc=False),
    out_type=jax.ShapeDtypeStruct((N, D), jnp.float32),
    scratch_types=[
        pltpu.VMEM((NCH, C), jnp.int32),
        pltpu.VMEM((NCH, C), jnp.int32),
        pltpu.VMEM((C, DH), jnp.float32),
        pltpu.VMEM((C, DH), jnp.float32),
        pltpu.VMEM((C, DH), jnp.float32),
        pltpu.VMEM((C, DH), jnp.float32),
        pltpu.VMEM_SHARED((N, DH), jnp.float32),
        pltpu.SemaphoreType.DMA,
        pltpu.SemaphoreType.DMA,
        pltpu.SemaphoreType.DMA,
        pltpu.SemaphoreType.DMA,
        pltpu.SemaphoreType.DMA,
        pltpu.SemaphoreType.DMA,
        pltpu.SemaphoreType.DMA,
        pltpu.SemaphoreType.DMA,
        pltpu.SemaphoreType.DMA,
        pltpu.SemaphoreType.DMA,
    ],
)
def _scatter_kernel(h_hbm, row_hbm, col_hbm, out_hbm,
                    rows_v, cols_v, g0, g1, g2, g3, acc,
                    gs0, gs1, gs2, gs3, ss0, ss1, ss2, ss3, is0, is1):
    c = lax.axis_index("c")
    s = lax.axis_index("s")
    base_r = s * RPT
    g = [g0, g1, g2, g3]
    gsem = [gs0, gs1, gs2, gs3]
    ssem = [ss0, ss1, ss2, ss3]

    icp0 = pltpu.async_copy(row_hbm.at[s], rows_v, is0)
    icp1 = pltpu.async_copy(col_hbm.at[s], cols_v, is1)

    def _zrow(i, carry):
        for t in range(DH // 16):
            g0[i, pl.ds(t * 16, 16)] = jnp.zeros((16,), jnp.float32)
        return carry
    lax.fori_loop(0, C, _zrow, 0)

    zcp = []
    for k in range(5):
        nr = 124 if k == 4 else 125
        zcp.append(pltpu.async_copy(
            g0.at[pl.ds(0, nr)],
            acc.at[pl.ds(base_r + k * 125, nr)],
            ssem[k % NBUF]))

    @pl.when(s == NSUB - 1)
    def _ztail():
        pltpu.async_copy(g[0].at[pl.ds(0, TAIL)],
                         acc.at[pl.ds(NSUB * RPT, TAIL)], ssem[0]).wait()

    for cp in zcp:
        cp.wait()
    icp0.wait()
    icp1.wait()

    plsc.subcore_barrier()

    for b in range(NBUF):
        pltpu.async_copy(h_hbm.at[c].at[cols_v.at[b]], g[b], gsem[b])

    def _group(grp, carry):
        j0 = grp * NBUF
        for b in range(NBUF):
            j = j0 + b
            pltpu.make_async_copy(h_hbm.at[c].at[cols_v.at[j]], g[b], gsem[b]).wait()
            pltpu.async_copy(g[b], acc.at[rows_v.at[j]], ssem[b], add=True)
        for b in range(NBUF):
            j = j0 + b
            nxt = j + NBUF

            @pl.when(nxt < NCH)
            def _refill():
                pltpu.make_async_copy(
                    g[b], acc.at[rows_v.at[j]], ssem[b]).wait()
                pltpu.async_copy(h_hbm.at[c].at[cols_v.at[nxt]], g[b], gsem[b])
        return carry
    lax.fori_loop(0, NCH // NBUF, _group, 0)

    for b in range(NBUF):
        j = NCH - NBUF + b
        pltpu.make_async_copy(g[b], acc.at[rows_v.at[j]], ssem[b]).wait()

    plsc.subcore_barrier()

    pltpu.sync_copy(acc.at[pl.ds(base_r, RPT)],
                    out_hbm.at[pl.ds(base_r, RPT), pl.ds(c * DH, DH)])

    @pl.when(s == NSUB - 1)
    def _wtail():
        pltpu.sync_copy(acc.at[pl.ds(NSUB * RPT, TAIL)],
                        out_hbm.at[pl.ds(NSUB * RPT, TAIL), pl.ds(c * DH, DH)])


def kernel(x, edge_index, W):
    h2 = _matmul(x, W)
    row3d = edge_index[0].reshape(NSUB, NCH, C)
    col3d = edge_index[1].reshape(NSUB, NCH, C)
    return _scatter_kernel(h2, row3d, col3d)


# speedup vs baseline: 1.2520x; 1.0334x over previous
import functools

import jax
import jax.numpy as jnp
from jax import lax
from jax.experimental import pallas as pl
from jax.experimental.pallas import tpu as pltpu
from jax.experimental.pallas import tpu_sc as plsc

N = 10000
E = 320000
D = 128
DH = D // 2

NCORES = 2
NSUB = 16
EPT = E // NSUB
C = 125
NCH = EPT // C
NBUF = 4
RPT = 624
TAIL = N - NSUB * RPT


def _mm_body(x_ref, w_ref, h_ref):
    h_ref[0] = lax.dot_general(
        x_ref[...], w_ref[...],
        (((1,), (1,)), ((), ())),
        preferred_element_type=jnp.float32,
    )


def _matmul(x, W):
    return pl.pallas_call(
        _mm_body,
        grid=(2, 5),
        in_specs=[
            pl.BlockSpec((N // 5, D), lambda k, i: (i, 0)),
            pl.BlockSpec((DH, D), lambda k, i: (k, 0)),
        ],
        out_specs=pl.BlockSpec((1, N // 5, DH), lambda k, i: (k, i, 0)),
        out_shape=jax.ShapeDtypeStruct((2, N, DH), jnp.float32),
    )(x, W)


def _cat_body(p_ref, o_ref):
    o_ref[...] = jnp.concatenate([p_ref[0], p_ref[1]], axis=-1)


def _assemble(p):
    return pl.pallas_call(
        _cat_body,
        grid=(10,),
        in_specs=[pl.BlockSpec((2, N // 10, DH), lambda i: (0, i, 0))],
        out_specs=pl.BlockSpec((N // 10, D), lambda i: (i, 0)),
        out_shape=jax.ShapeDtypeStruct((N, D), jnp.float32),
    )(p)


_mesh = plsc.VectorSubcoreMesh(core_axis_name="c", subcore_axis_name="s")


@functools.partial(
    pl.kernel,
    mesh=_mesh,
    compiler_params=pltpu.CompilerParams(use_tc_tiling_on_sc=False),
    out_type=jax.ShapeDtypeStruct((N, D), jnp.float32),
    scratch_types=[
        pltpu.VMEM((NCH, C), jnp.int32),
        pltpu.VMEM((NCH, C), jnp.int32),
        pltpu.VMEM((C, DH), jnp.float32),
        pltpu.VMEM((C, DH), jnp.float32),
        pltpu.VMEM((C, DH), jnp.float32),
        pltpu.VMEM((C, DH), jnp.float32),
        pltpu.VMEM_SHARED((N, DH), jnp.float32),
        pltpu.SemaphoreType.DMA,
        pltpu.SemaphoreType.DMA,
        pltpu.SemaphoreType.DMA,
        pltpu.SemaphoreType.DMA,
        pltpu.SemaphoreType.DMA,
        pltpu.SemaphoreType.DMA,
        pltpu.SemaphoreType.DMA,
        pltpu.SemaphoreType.DMA,
        pltpu.SemaphoreType.DMA,
        pltpu.SemaphoreType.DMA,
    ],
)
def _scatter_kernel(h_hbm, row_hbm, col_hbm, out_hbm,
                    rows_v, cols_v, g0, g1, g2, g3, acc,
                    gs0, gs1, gs2, gs3, ss0, ss1, ss2, ss3, is0, is1):
    c = lax.axis_index("c")
    s = lax.axis_index("s")
    base_r = s * RPT
    g = [g0, g1, g2, g3]
    gsem = [gs0, gs1, gs2, gs3]
    ssem = [ss0, ss1, ss2, ss3]

    icp0 = pltpu.async_copy(row_hbm.at[s], rows_v, is0)
    icp1 = pltpu.async_copy(col_hbm.at[s], cols_v, is1)

    def _zrow(i, carry):
        for t in range(DH // 16):
            g0[i, pl.ds(t * 16, 16)] = jnp.zeros((16,), jnp.float32)
        return carry
    lax.fori_loop(0, C, _zrow, 0)

    zcp = []
    for k in range(5):
        nr = 124 if k == 4 else 125
        zcp.append(pltpu.async_copy(
            g0.at[pl.ds(0, nr)],
            acc.at[pl.ds(base_r + k * 125, nr)],
            ssem[k % NBUF]))

    @pl.when(s == NSUB - 1)
    def _ztail():
        pltpu.async_copy(g[0].at[pl.ds(0, TAIL)],
                         acc.at[pl.ds(NSUB * RPT, TAIL)], ssem[0]).wait()

    for cp in zcp:
        cp.wait()
    icp0.wait()
    icp1.wait()

    plsc.subcore_barrier()

    for b in range(NBUF):
        pltpu.async_copy(h_hbm.at[c].at[cols_v.at[b]], g[b], gsem[b])

    def _group(grp, carry):
        j0 = grp * NBUF
        for b in range(NBUF):
            j = j0 + b
            pltpu.make_async_copy(h_hbm.at[c].at[cols_v.at[j]], g[b], gsem[b]).wait()
            pltpu.async_copy(g[b], acc.at[rows_v.at[j]], ssem[b], add=True)
        for b in range(NBUF):
            j = j0 + b
            nxt = j + NBUF

            @pl.when(nxt < NCH)
            def _refill():
                pltpu.make_async_copy(
                    g[b], acc.at[rows_v.at[j]], ssem[b]).wait()
                pltpu.async_copy(h_hbm.at[c].at[cols_v.at[nxt]], g[b], gsem[b])
        return carry
    lax.fori_loop(0, NCH // NBUF, _group, 0)

    for b in range(NBUF):
        j = NCH - NBUF + b
        pltpu.make_async_copy(g[b], acc.at[rows_v.at[j]], ssem[b]).wait()

    plsc.subcore_barrier()

    pltpu.sync_copy(acc.at[pl.ds(base_r, RPT)],
                    out_hbm.at[pl.ds(base_r, RPT), pl.ds(c * DH, DH)])

    @pl.when(s == NSUB - 1)
    def _wtail():
        pltpu.sync_copy(acc.at[pl.ds(NSUB * RPT, TAIL)],
                        out_hbm.at[pl.ds(NSUB * RPT, TAIL), pl.ds(c * DH, DH)])


def kernel(x, edge_index, W):
    h2 = _matmul(x, W)
    row3d = edge_index[0].reshape(NSUB, NCH, C)
    col3d = edge_index[1].reshape(NSUB, NCH, C)
    return _scatter_kernel(h2, row3d, col3d)


# speedup vs baseline: 1.2812x; 1.0233x over previous
import functools

import jax
import jax.numpy as jnp
from jax import lax
from jax.experimental import pallas as pl
from jax.experimental.pallas import tpu as pltpu
from jax.experimental.pallas import tpu_sc as plsc

N = 10000
E = 320000
D = 128
DH = D // 2

NCORES = 2
NSUB = 16
EPT = E // NSUB
C = 125
NCH = EPT // C
NBUF = 4
RPT = 624
TAIL = N - NSUB * RPT


def _mm_body(x_ref, w_ref, h_ref):
    x = x_ref[...]
    for k in range(2):
        h_ref[k] = lax.dot_general(
            x, w_ref[pl.ds(k * DH, DH), :],
            (((1,), (1,)), ((), ())),
            preferred_element_type=jnp.float32,
        )


def _matmul(x, W):
    return pl.pallas_call(
        _mm_body,
        grid=(5,),
        in_specs=[
            pl.BlockSpec((N // 5, D), lambda i: (i, 0)),
            pl.BlockSpec((D, D), lambda i: (0, 0)),
        ],
        out_specs=pl.BlockSpec((2, N // 5, DH), lambda i: (0, i, 0)),
        out_shape=jax.ShapeDtypeStruct((2, N, DH), jnp.float32),
    )(x, W)


def _cat_body(p_ref, o_ref):
    o_ref[...] = jnp.concatenate([p_ref[0], p_ref[1]], axis=-1)


def _assemble(p):
    return pl.pallas_call(
        _cat_body,
        grid=(10,),
        in_specs=[pl.BlockSpec((2, N // 10, DH), lambda i: (0, i, 0))],
        out_specs=pl.BlockSpec((N // 10, D), lambda i: (i, 0)),
        out_shape=jax.ShapeDtypeStruct((N, D), jnp.float32),
    )(p)


_mesh = plsc.VectorSubcoreMesh(core_axis_name="c", subcore_axis_name="s")


@functools.partial(
    pl.kernel,
    mesh=_mesh,
    compiler_params=pltpu.CompilerParams(use_tc_tiling_on_sc=False),
    out_type=jax.ShapeDtypeStruct((N, D), jnp.float32),
    scratch_types=[
        pltpu.VMEM((NCH, C), jnp.int32),
        pltpu.VMEM((NCH, C), jnp.int32),
        pltpu.VMEM((C, DH), jnp.float32),
        pltpu.VMEM((C, DH), jnp.float32),
        pltpu.VMEM((C, DH), jnp.float32),
        pltpu.VMEM((C, DH), jnp.float32),
        pltpu.VMEM_SHARED((N, DH), jnp.float32),
        pltpu.SemaphoreType.DMA,
        pltpu.SemaphoreType.DMA,
        pltpu.SemaphoreType.DMA,
        pltpu.SemaphoreType.DMA,
        pltpu.SemaphoreType.DMA,
        pltpu.SemaphoreType.DMA,
        pltpu.SemaphoreType.DMA,
        pltpu.SemaphoreType.DMA,
        pltpu.SemaphoreType.DMA,
        pltpu.SemaphoreType.DMA,
    ],
)
def _scatter_kernel(h_hbm, row_hbm, col_hbm, out_hbm,
                    rows_v, cols_v, g0, g1, g2, g3, acc,
                    gs0, gs1, gs2, gs3, ss0, ss1, ss2, ss3, is0, is1):
    c = lax.axis_index("c")
    s = lax.axis_index("s")
    base_r = s * RPT
    g = [g0, g1, g2, g3]
    gsem = [gs0, gs1, gs2, gs3]
    ssem = [ss0, ss1, ss2, ss3]

    icp0 = pltpu.async_copy(row_hbm.at[s], rows_v, is0)
    icp1 = pltpu.async_copy(col_hbm.at[s], cols_v, is1)

    def _zrow(i, carry):
        for t in range(DH // 16):
            g0[i, pl.ds(t * 16, 16)] = jnp.zeros((16,), jnp.float32)
        return carry
    lax.fori_loop(0, C, _zrow, 0)

    zcp = []
    for k in range(5):
        nr = 124 if k == 4 else 125
        zcp.append(pltpu.async_copy(
            g0.at[pl.ds(0, nr)],
            acc.at[pl.ds(base_r + k * 125, nr)],
            ssem[k % NBUF]))

    @pl.when(s == NSUB - 1)
    def _ztail():
        pltpu.async_copy(g[0].at[pl.ds(0, TAIL)],
                         acc.at[pl.ds(NSUB * RPT, TAIL)], ssem[0]).wait()

    for cp in zcp:
        cp.wait()
    icp0.wait()
    icp1.wait()

    plsc.subcore_barrier()

    for b in range(NBUF):
        pltpu.async_copy(h_hbm.at[c].at[cols_v.at[b]], g[b], gsem[b])

    def _group(grp, carry):
        j0 = grp * NBUF
        for b in range(NBUF):
            j = j0 + b
            pltpu.make_async_copy(h_hbm.at[c].at[cols_v.at[j]], g[b], gsem[b]).wait()
            pltpu.async_copy(g[b], acc.at[rows_v.at[j]], ssem[b], add=True)
        for b in range(NBUF):
            j = j0 + b
            nxt = j + NBUF

            @pl.when(nxt < NCH)
            def _refill():
                pltpu.make_async_copy(
                    g[b], acc.at[rows_v.at[j]], ssem[b]).wait()
                pltpu.async_copy(h_hbm.at[c].at[cols_v.at[nxt]], g[b], gsem[b])
        return carry
    lax.fori_loop(0, NCH // NBUF, _group, 0)

    for b in range(NBUF):
        j = NCH - NBUF + b
        pltpu.make_async_copy(g[b], acc.at[rows_v.at[j]], ssem[b]).wait()

    plsc.subcore_barrier()

    pltpu.sync_copy(acc.at[pl.ds(base_r, RPT)],
                    out_hbm.at[pl.ds(base_r, RPT), pl.ds(c * DH, DH)])

    @pl.when(s == NSUB - 1)
    def _wtail():
        pltpu.sync_copy(acc.at[pl.ds(NSUB * RPT, TAIL)],
                        out_hbm.at[pl.ds(NSUB * RPT, TAIL), pl.ds(c * DH, DH)])


def kernel(x, edge_index, W):
    h2 = _matmul(x, W)
    row3d = edge_index[0].reshape(NSUB, NCH, C)
    col3d = edge_index[1].reshape(NSUB, NCH, C)
    return _scatter_kernel(h2, row3d, col3d)
